# baseline (speedup 1.0000x reference)
import functools

import jax
import jax.numpy as jnp
from jax.experimental import pallas as pl
from jax.experimental.pallas import tpu as pltpu

_N = {"ind": 100000, "org": 20000, "ext": 5000}
_RELS1 = [
    ("ind", "txn", "ind"), ("org", "txn", "ind"), ("ext", "txn", "ind"),
    ("ind", "txn", "org"), ("org", "txn", "org"), ("ext", "txn", "org"),
    ("ind", "txn", "ext"), ("org", "txn", "ext"), ("ind", "role", "org"),
    ("ind", "rev_txn", "ind"), ("org", "rev_txn", "ind"), ("ext", "rev_txn", "ind"),
    ("ind", "rev_txn", "org"), ("org", "rev_txn", "org"), ("ext", "rev_txn", "org"),
    ("ind", "rev_txn", "ext"), ("org", "rev_txn", "ext"), ("org", "rev_role", "ind"),
]
_RELS2 = [r for r in _RELS1 if r[2] != "ext"]
_TYPES = ["ind", "org", "ext"]

_SPLIT_ROWS = 50000


def _key(r):
    return r[0] + "__" + r[1] + "__" + r[2]


def _agg_body(src_ref, dst_ref, x_ref, out_ref, *,
              n_edges, s_split, d_split, s_par, d_par):
    out_ref[...] = jnp.zeros_like(out_ref)
    n_rows = x_ref.shape[0]

    def body(i, carry):
        s = src_ref[i]
        d = dst_ref[i]
        ok_s = (s % s_split) == s_par
        ok_d = (d % d_split) == d_par

        @pl.when(jnp.logical_and(ok_s, ok_d))
        def _():
            sh = jnp.minimum(s // s_split, n_rows - 1)
            dh = d // d_split
            out_ref[pl.ds(dh, 1), :] = (
                out_ref[pl.ds(dh, 1), :] + x_ref[pl.ds(sh, 1), :])

        return carry

    jax.lax.fori_loop(0, n_edges, body, 0)


@functools.partial(jax.jit, static_argnums=(3, 4, 5, 6, 7))
def _agg_pass(src, dst, x, n_out, s_split, d_split, s_par, d_par):
    n_edges = src.shape[0]
    n_src, width = x.shape
    return pl.pallas_call(
        functools.partial(_agg_body, n_edges=n_edges, s_split=s_split,
                          d_split=d_split, s_par=s_par, d_par=d_par),
        grid_spec=pltpu.PrefetchScalarGridSpec(
            num_scalar_prefetch=2,
            grid=(1,),
            in_specs=[
                pl.BlockSpec((n_src, width), lambda i, s, d: (0, 0)),
            ],
            out_specs=pl.BlockSpec((n_out, width), lambda i, s, d: (0, 0)),
        ),
        out_shape=jax.ShapeDtypeStruct((n_out, width), jnp.float32),
    )(src, dst, x)


def _segment_sum(src, dst, x, n_dst):
    n_src, width = x.shape
    s_split = 2 if (n_src > _SPLIT_ROWS and width > 1) else 1
    d_split = 2 if (n_dst > _SPLIT_ROWS and width > 1) else 1
    x_halves = [x] if s_split == 1 else [x[0::2], x[1::2]]
    d_outs = []
    for d_par in range(d_split):
        acc = None
        for s_par, xh in enumerate(x_halves):
            o = _agg_pass(src, dst, xh, n_dst // d_split,
                          s_split, d_split, s_par, d_par)
            acc = o if acc is None else acc + o
        d_outs.append(acc)
    if d_split == 1:
        return d_outs[0]
    return jnp.stack(d_outs, axis=1).reshape(n_dst, width)


def _dense1_body(*refs, n_rel):
    s_refs = refs[:n_rel]
    c_ref, x_ref, wl_ref, wr_ref, b_ref, h_ref = refs[n_rel:]
    acc = jnp.dot(x_ref[...], wr_ref[...], preferred_element_type=jnp.float32)
    acc = acc + b_ref[0, :][None, :]
    for r in range(n_rel):
        cnt = jnp.maximum(c_ref[:, r], 1.0)[:, None]
        mean = s_refs[r][...] / cnt
        acc = acc + jnp.dot(mean, wl_ref[r], preferred_element_type=jnp.float32)
    h_ref[...] = jax.nn.sigmoid(acc)


@functools.partial(jax.jit, static_argnums=(6,))
def _dense1(s_list, c_all, x, wl_stack, wr_sum, bias, block):
    n, d = x.shape
    n_rel = len(s_list)
    grid = (pl.cdiv(n, block),)
    in_specs = (
        [pl.BlockSpec((block, d), lambda i: (i, 0)) for _ in range(n_rel)]
        + [
            pl.BlockSpec((block, n_rel), lambda i: (i, 0)),
            pl.BlockSpec((block, d), lambda i: (i, 0)),
            pl.BlockSpec((n_rel, d, d), lambda i: (0, 0, 0)),
            pl.BlockSpec((d, d), lambda i: (0, 0)),
            pl.BlockSpec((1, d), lambda i: (0, 0)),
        ]
    )
    return pl.pallas_call(
        functools.partial(_dense1_body, n_rel=n_rel),
        grid=grid,
        in_specs=in_specs,
        out_specs=pl.BlockSpec((block, d), lambda i: (i, 0)),
        out_shape=jax.ShapeDtypeStruct((n, d), jnp.float32),
    )(*s_list, c_all, x, wl_stack, wr_sum, bias)


def _dense2_body(*refs, n_rel):
    s_refs = refs[:n_rel]
    c_ref, h_ref, wl_ref, wr_ref, b_ref, o_ref = refs[n_rel:]
    acc = jnp.dot(h_ref[...], wr_ref[...], preferred_element_type=jnp.float32)
    acc = acc + b_ref[0, 0]
    for r in range(n_rel):
        cnt = jnp.maximum(c_ref[:, r], 1.0)[:, None]
        mean = s_refs[r][...] / cnt
        acc = acc + jnp.dot(mean, wl_ref[r], preferred_element_type=jnp.float32)
    o_ref[...] = jax.nn.sigmoid(acc)


@functools.partial(jax.jit, static_argnums=(6,))
def _dense2(s_list, c_all, h, wl_stack, wr_sum, bias, block):
    n, d = h.shape
    n_rel = len(s_list)
    grid = (pl.cdiv(n, block),)
    in_specs = (
        [pl.BlockSpec((block, d), lambda i: (i, 0)) for _ in range(n_rel)]
        + [
            pl.BlockSpec((block, n_rel), lambda i: (i, 0)),
            pl.BlockSpec((block, d), lambda i: (i, 0)),
            pl.BlockSpec((n_rel, d, 1), lambda i: (0, 0, 0)),
            pl.BlockSpec((d, 1), lambda i: (0, 0)),
            pl.BlockSpec((1, 1), lambda i: (0, 0)),
        ]
    )
    return pl.pallas_call(
        functools.partial(_dense2_body, n_rel=n_rel),
        grid=grid,
        in_specs=in_specs,
        out_specs=pl.BlockSpec((block, 1), lambda i: (i, 0)),
        out_shape=jax.ShapeDtypeStruct((n, 1), jnp.float32),
    )(*s_list, c_all, h, wl_stack, wr_sum, bias)


def kernel(x_dict, edge_index_dict, edge_attr_dict, params1, params2):
    del edge_attr_dict

    ones_table = jnp.ones((8, 1), jnp.float32)

    seg = {}
    cnt = {}
    for r in _RELS1:
        k = _key(r)
        ei = edge_index_dict[k]
        n_dst = _N[r[2]]
        seg[k] = _segment_sum(ei[0], ei[1], x_dict[r[0]], n_dst)
        cnt[k] = _segment_sum(ei[1], ei[1], ones_table, n_dst)

    h = {}
    for t in _TYPES:
        rels_t = [r for r in _RELS1 if r[2] == t]
        s_list = [seg[_key(r)] for r in rels_t]
        c_all = jnp.concatenate([cnt[_key(r)] for r in rels_t], axis=1)
        wl_stack = jnp.stack([params1[_key(r)]["Wl"] for r in rels_t], axis=0)
        wr_sum = sum(params1[_key(r)]["Wr"] for r in rels_t)
        bias = sum(params1[_key(r)]["bl"] for r in rels_t)[None, :]
        block = 512 if t != "ext" else 256
        h[t] = _dense1(s_list, c_all, x_dict[t], wl_stack, wr_sum, bias, block)

    seg2 = {}
    for r in _RELS2:
        k = _key(r)
        ei = edge_index_dict[k]
        seg2[k] = _segment_sum(ei[0], ei[1], h[r[0]], _N[r[2]])

    preds = []
    for t in ["ind", "org"]:
        rels_t = [r for r in _RELS2 if r[2] == t]
        s_list = [seg2[_key(r)] for r in rels_t]
        c_all = jnp.concatenate([cnt[_key(r)] for r in rels_t], axis=1)
        wl_stack = jnp.stack([params2[_key(r)]["Wl"] for r in rels_t], axis=0)
        wr_sum = sum(params2[_key(r)]["Wr"] for r in _RELS2 if r[2] == t)
        bias = sum(params2[_key(r)]["bl"] for r in rels_t)[None, None, 0]
        preds.append(_dense2(s_list, c_all, h[t], wl_stack, wr_sum, bias, 512)[:, 0])

    return (preds[0], preds[1])


# speedup vs baseline: 1.2994x; 1.2994x over previous
import functools

import jax
import jax.numpy as jnp
from jax.experimental import pallas as pl
from jax.experimental.pallas import tpu as pltpu

_N = {"ind": 100000, "org": 20000, "ext": 5000}
_RELS1 = [
    ("ind", "txn", "ind"), ("org", "txn", "ind"), ("ext", "txn", "ind"),
    ("ind", "txn", "org"), ("org", "txn", "org"), ("ext", "txn", "org"),
    ("ind", "txn", "ext"), ("org", "txn", "ext"), ("ind", "role", "org"),
    ("ind", "rev_txn", "ind"), ("org", "rev_txn", "ind"), ("ext", "rev_txn", "ind"),
    ("ind", "rev_txn", "org"), ("org", "rev_txn", "org"), ("ext", "rev_txn", "org"),
    ("ind", "rev_txn", "ext"), ("org", "rev_txn", "ext"), ("org", "rev_role", "ind"),
]
_RELS2 = [r for r in _RELS1 if r[2] != "ext"]
_TYPES = ["ind", "org", "ext"]

_SPLIT_ROWS = 50000


def _key(r):
    return r[0] + "__" + r[1] + "__" + r[2]


def _agg_body(src_ref, dst_ref, x_ref, out_ref, *,
              n_edges, s_split, d_split, s_par, d_par):
    out_ref[...] = jnp.zeros_like(out_ref)
    n_rows = x_ref.shape[0]

    def body(i, carry):
        s = src_ref[i]
        d = dst_ref[i]
        ok_s = (s % s_split) == s_par
        ok_d = (d % d_split) == d_par

        @pl.when(jnp.logical_and(ok_s, ok_d))
        def _():
            sh = jnp.minimum(s // s_split, n_rows - 1)
            dh = d // d_split
            out_ref[pl.ds(dh, 1), :] = (
                out_ref[pl.ds(dh, 1), :] + x_ref[pl.ds(sh, 1), :])

        return carry

    jax.lax.fori_loop(0, n_edges, body, 0, unroll=8)


@functools.partial(jax.jit, static_argnums=(3, 4, 5, 6, 7))
def _agg_pass(src, dst, x, n_out, s_split, d_split, s_par, d_par):
    n_edges = src.shape[0]
    n_src, width = x.shape
    return pl.pallas_call(
        functools.partial(_agg_body, n_edges=n_edges, s_split=s_split,
                          d_split=d_split, s_par=s_par, d_par=d_par),
        grid_spec=pltpu.PrefetchScalarGridSpec(
            num_scalar_prefetch=2,
            grid=(1,),
            in_specs=[
                pl.BlockSpec((n_src, width), lambda i, s, d: (0, 0)),
            ],
            out_specs=pl.BlockSpec((n_out, width), lambda i, s, d: (0, 0)),
        ),
        out_shape=jax.ShapeDtypeStruct((n_out, width), jnp.float32),
    )(src, dst, x)


def _segment_sum(src, dst, x, n_dst):
    n_src, width = x.shape
    s_split = 2 if (n_src > _SPLIT_ROWS and width > 1) else 1
    d_split = 2 if (n_dst > _SPLIT_ROWS and width > 1) else 1
    x_halves = [x] if s_split == 1 else [x[0::2], x[1::2]]
    d_outs = []
    for d_par in range(d_split):
        acc = None
        for s_par, xh in enumerate(x_halves):
            o = _agg_pass(src, dst, xh, n_dst // d_split,
                          s_split, d_split, s_par, d_par)
            acc = o if acc is None else acc + o
        d_outs.append(acc)
    if d_split == 1:
        return d_outs[0]
    return jnp.stack(d_outs, axis=1).reshape(n_dst, width)


def _dense1_body(*refs, n_rel):
    s_refs = refs[:n_rel]
    c_ref, x_ref, wl_ref, wr_ref, b_ref, h_ref = refs[n_rel:]
    acc = jnp.dot(x_ref[...], wr_ref[...], preferred_element_type=jnp.float32)
    acc = acc + b_ref[0, :][None, :]
    for r in range(n_rel):
        cnt = jnp.maximum(c_ref[:, r], 1.0)[:, None]
        mean = s_refs[r][...] / cnt
        acc = acc + jnp.dot(mean, wl_ref[r], preferred_element_type=jnp.float32)
    h_ref[...] = jax.nn.sigmoid(acc)


@functools.partial(jax.jit, static_argnums=(6,))
def _dense1(s_list, c_all, x, wl_stack, wr_sum, bias, block):
    n, d = x.shape
    n_rel = len(s_list)
    grid = (pl.cdiv(n, block),)
    in_specs = (
        [pl.BlockSpec((block, d), lambda i: (i, 0)) for _ in range(n_rel)]
        + [
            pl.BlockSpec((block, n_rel), lambda i: (i, 0)),
            pl.BlockSpec((block, d), lambda i: (i, 0)),
            pl.BlockSpec((n_rel, d, d), lambda i: (0, 0, 0)),
            pl.BlockSpec((d, d), lambda i: (0, 0)),
            pl.BlockSpec((1, d), lambda i: (0, 0)),
        ]
    )
    return pl.pallas_call(
        functools.partial(_dense1_body, n_rel=n_rel),
        grid=grid,
        in_specs=in_specs,
        out_specs=pl.BlockSpec((block, d), lambda i: (i, 0)),
        out_shape=jax.ShapeDtypeStruct((n, d), jnp.float32),
    )(*s_list, c_all, x, wl_stack, wr_sum, bias)


def _dense2_body(*refs, n_rel):
    s_refs = refs[:n_rel]
    c_ref, h_ref, wl_ref, wr_ref, b_ref, o_ref = refs[n_rel:]
    acc = jnp.dot(h_ref[...], wr_ref[...], preferred_element_type=jnp.float32)
    acc = acc + b_ref[0, 0]
    for r in range(n_rel):
        cnt = jnp.maximum(c_ref[:, r], 1.0)[:, None]
        mean = s_refs[r][...] / cnt
        acc = acc + jnp.dot(mean, wl_ref[r], preferred_element_type=jnp.float32)
    o_ref[...] = jax.nn.sigmoid(acc)


@functools.partial(jax.jit, static_argnums=(6,))
def _dense2(s_list, c_all, h, wl_stack, wr_sum, bias, block):
    n, d = h.shape
    n_rel = len(s_list)
    grid = (pl.cdiv(n, block),)
    in_specs = (
        [pl.BlockSpec((block, d), lambda i: (i, 0)) for _ in range(n_rel)]
        + [
            pl.BlockSpec((block, n_rel), lambda i: (i, 0)),
            pl.BlockSpec((block, d), lambda i: (i, 0)),
            pl.BlockSpec((n_rel, d, 1), lambda i: (0, 0, 0)),
            pl.BlockSpec((d, 1), lambda i: (0, 0)),
            pl.BlockSpec((1, 1), lambda i: (0, 0)),
        ]
    )
    return pl.pallas_call(
        functools.partial(_dense2_body, n_rel=n_rel),
        grid=grid,
        in_specs=in_specs,
        out_specs=pl.BlockSpec((block, 1), lambda i: (i, 0)),
        out_shape=jax.ShapeDtypeStruct((n, 1), jnp.float32),
    )(*s_list, c_all, h, wl_stack, wr_sum, bias)


def kernel(x_dict, edge_index_dict, edge_attr_dict, params1, params2):
    del edge_attr_dict

    ones_table = jnp.ones((8, 1), jnp.float32)

    seg = {}
    cnt = {}
    for r in _RELS1:
        k = _key(r)
        ei = edge_index_dict[k]
        n_dst = _N[r[2]]
        seg[k] = _segment_sum(ei[0], ei[1], x_dict[r[0]], n_dst)
        cnt[k] = _segment_sum(ei[1], ei[1], ones_table, n_dst)

    h = {}
    for t in _TYPES:
        rels_t = [r for r in _RELS1 if r[2] == t]
        s_list = [seg[_key(r)] for r in rels_t]
        c_all = jnp.concatenate([cnt[_key(r)] for r in rels_t], axis=1)
        wl_stack = jnp.stack([params1[_key(r)]["Wl"] for r in rels_t], axis=0)
        wr_sum = sum(params1[_key(r)]["Wr"] for r in rels_t)
        bias = sum(params1[_key(r)]["bl"] for r in rels_t)[None, :]
        block = 512 if t != "ext" else 256
        h[t] = _dense1(s_list, c_all, x_dict[t], wl_stack, wr_sum, bias, block)

    seg2 = {}
    for r in _RELS2:
        k = _key(r)
        ei = edge_index_dict[k]
        seg2[k] = _segment_sum(ei[0], ei[1], h[r[0]], _N[r[2]])

    preds = []
    for t in ["ind", "org"]:
        rels_t = [r for r in _RELS2 if r[2] == t]
        s_list = [seg2[_key(r)] for r in rels_t]
        c_all = jnp.concatenate([cnt[_key(r)] for r in rels_t], axis=1)
        wl_stack = jnp.stack([params2[_key(r)]["Wl"] for r in rels_t], axis=0)
        wr_sum = sum(params2[_key(r)]["Wr"] for r in _RELS2 if r[2] == t)
        bias = sum(params2[_key(r)]["bl"] for r in rels_t)[None, None, 0]
        preds.append(_dense2(s_list, c_all, h[t], wl_stack, wr_sum, bias, 512)[:, 0])

    return (preds[0], preds[1])
